# baseline (speedup 1.0000x reference)
import functools

import jax
import jax.numpy as jnp
from jax.experimental import pallas as pl

BATCH = 16384
FIELD_DIM = 1000
EMBED_DIM = 64
BLOCK_K = 200


def _embed_block(idx_ref, w_ref, out_ref):
    mask = (idx_ref[...] != 0).astype(jnp.float32)
    wt = w_ref[...].T
    part = jnp.dot(wt, mask, preferred_element_type=jnp.float32)

    @pl.when(pl.program_id(0) == 0)
    def _init():
        out_ref[...] = part

    @pl.when(pl.program_id(0) != 0)
    def _acc():
        out_ref[...] += part


@functools.partial(jax.jit, static_argnames=())
def kernel(indices, W):
    idx_t = indices.T
    out_t = pl.pallas_call(
        _embed_block,
        grid=(FIELD_DIM // BLOCK_K,),
        in_specs=[
            pl.BlockSpec((BLOCK_K, BATCH), lambda k: (k, 0)),
            pl.BlockSpec((BLOCK_K, EMBED_DIM), lambda k: (k, 0)),
        ],
        out_specs=pl.BlockSpec((EMBED_DIM, BATCH), lambda k: (0, 0)),
        out_shape=jax.ShapeDtypeStruct((EMBED_DIM, BATCH), jnp.float32),
    )(idx_t, W)
    return out_t.T
